# baseline (speedup 1.0000x reference)
import functools

import jax
import jax.numpy as jnp
from jax import lax
from jax.experimental import pallas as pl
from jax.experimental.pallas import tpu as pltpu
from jax.experimental.pallas import tpu_sc as plsc

B = 8192
N = 64
NUM_ATOM = 16
NUM_EDGE = 5
MAXN = 64
VAL_LEN = 3 * MAXN - 2

OFF_N = 0
OFF_NODE = 80
OFF_EDGE = 96
OFF_VAL = 112
HTOT = 304

NC = 2
NS = 16
NW = NC * NS
MPW = B // NW
L = 16
LB = 128
JW = 8


def _sc_histograms(atom_t, edge_t, n_nodes):
    mesh = plsc.VectorSubcoreMesh(
        core_axis_name="c", subcore_axis_name="s", num_cores=NC, num_subcores=NS
    )

    @functools.partial(
        pl.kernel,
        mesh=mesh,
        compiler_params=pltpu.CompilerParams(needs_layout_passes=False),
        out_type=jax.ShapeDtypeStruct((NW, HTOT), jnp.float32),
        scratch_types=[
            pltpu.VMEM((MPW,), jnp.int32),
            pltpu.VMEM((N, LB), jnp.int32),
            pltpu.VMEM((N, JW, LB), jnp.int32),
            pltpu.VMEM((HTOT,), jnp.float32),
        ],
    )
    def body(atom_hbm, edge_hbm, n_hbm, out_hbm, nbuf, abuf, ebuf, hist):
        wid = lax.axis_index("s") * NC + lax.axis_index("c")
        base = wid * MPW

        zf = jnp.zeros((L,), jnp.float32)
        for h in range(HTOT // L):
            hist[pl.ds(h * L, L)] = zf

        pltpu.sync_copy(n_hbm.at[pl.ds(base, MPW)], nbuf)

        iota = lax.iota(jnp.int32, L)
        onesf = jnp.ones((L,), jnp.float32)
        zi = jnp.zeros((L,), jnp.int32)

        def ngrp(g, c):
            nv = nbuf[pl.ds(g * L, L)]
            plsc.addupdate_scatter(hist, [nv + OFF_N], onesf)
            return c
        lax.fori_loop(0, MPW // L, ngrp, 0)

        for mg in range(MPW // LB):
            mb = base + mg * LB

            pltpu.sync_copy(atom_hbm.at[:, pl.ds(mb, LB)], abuf)
            for sg in range(LB // L):
                n_vec = nbuf[pl.ds(mg * LB + sg * L, L)]
                maxn = jnp.max(n_vec)

                def node_row(i, cc, sg=sg, n_vec=n_vec):
                    av = abuf[i, pl.ds(sg * L, L)]
                    plsc.addupdate_scatter(
                        hist, [av + OFF_NODE], onesf, mask=i < n_vec
                    )
                    return cc
                lax.fori_loop(0, maxn, node_row, 0)

            def jslab(jc, c, mb=mb, mg=mg):
                pltpu.sync_copy(
                    edge_hbm.at[:, pl.ds(jc * JW, JW), pl.ds(mb, LB)], ebuf
                )
                fb16 = jnp.zeros((2 * L,), jnp.bool_)
                z16 = jnp.zeros((2 * L,), jnp.int16)
                j0 = jc * JW
                for p in range(LB // (2 * L)):
                    la = mg * LB + p * 2 * L
                    n_a = nbuf[pl.ds(la, L)]
                    n_b = nbuf[pl.ds(la + L, L)]
                    maxn = jnp.maximum(jnp.max(n_a), jnp.max(n_b))
                    n16 = plsc.pack(n_a, n_b, format=plsc.PackFormat.INTERLEAVED)

                    @pl.when(j0 < maxn)
                    def _(p=p, n_a=n_a, n_b=n_b, n16=n16, maxn=maxn, jc=jc, j0=j0):
                        mjs = [n16 > (j0 + jl).astype(jnp.int16)
                               for jl in range(JW)]

                        def load16(i, jl, p=p):
                            va_ = ebuf[i, jl, pl.ds(p * 2 * L, L)]
                            vb_ = ebuf[i, jl, pl.ds(p * 2 * L + L, L)]
                            return plsc.pack(
                                va_, vb_, format=plsc.PackFormat.INTERLEAVED
                            )

                        def body(i, carry, band):
                            vas, accs = carry
                            i16s = i.astype(jnp.int16)
                            iv = z16 + i16s
                            mi = n16 > i16s
                            vas, accs = list(vas), list(accs)
                            for jl in range(JW):
                                v = load16(i, jl)
                                etv = jnp.where(v == jnp.int16(4),
                                                jnp.int16(1), v)
                                pm = mi & mjs[jl]
                                vas[jl] = vas[jl] + jnp.where(pm, etv, z16)
                                if band == "tri":
                                    em = pm
                                elif band == "mix":
                                    em = pm & (iv < (j0 + jl).astype(jnp.int16))
                                else:
                                    continue
                                for e in range(1, NUM_EDGE):
                                    ce = em & (v == jnp.int16(e))
                                    accs[e - 1] = accs[e - 1] + jnp.where(
                                        ce, v, z16)
                            return tuple(vas), tuple(accs)

                        init = ((z16,) * JW, (z16,) * (NUM_EDGE - 1))
                        i_tri = jnp.minimum(j0, maxn)
                        i_mix = jnp.minimum(j0 + JW, maxn)
                        carry = lax.fori_loop(
                            0, i_tri, lambda i, ca: body(i, ca, "tri"), init)
                        carry = lax.fori_loop(
                            i_tri, i_mix, lambda i, ca: body(i, ca, "mix"), carry)
                        carry = lax.fori_loop(
                            i_mix, maxn, lambda i, ca: body(i, ca, "val"), carry)
                        vas, accs = carry

                        for jl in range(JW):
                            a_, b_ = plsc.unpack(
                                vas[jl], format=plsc.PackFormat.INTERLEAVED
                            )
                            for half, nv in ((a_, n_a), (b_, n_b)):
                                vc = (jnp.minimum(half.astype(jnp.int32),
                                                  VAL_LEN - 1) + OFF_VAL)
                                plsc.addupdate_scatter(
                                    hist, [vc], onesf, mask=j0 + jl < nv
                                )

                        sv = jnp.zeros((L,), jnp.float32)
                        for e in range(1, NUM_EDGE):
                            w = plsc.bitcast(accs[e - 1], jnp.int32)
                            t = jnp.sum(w)
                            s = ((t & 0xFFFF) + (t >> 16)) // e
                            sv = sv + jnp.where(
                                iota == e - 1, s.astype(jnp.float32), zf)
                        plsc.addupdate_scatter(hist, [iota + OFF_EDGE + 1], sv)
                return c

            lax.fori_loop(0, N // JW, jslab, 0)

        pltpu.sync_copy(hist, out_hbm.at[wid])

    return body(atom_t, edge_t, n_nodes)


def _finish_body(part_ref, nt_ref, node_t_ref, et_ref,
                 n_out, node_out, edge_out, val_out,
                 nmae_out, node_mae_out, edge_mae_out):
    part = part_ref[...]

    n_hist = jnp.sum(part[:, OFF_N:OFF_N + MAXN + 1], axis=0, keepdims=True)
    node_hist = jnp.sum(part[:, OFF_NODE:OFF_NODE + NUM_ATOM], axis=0, keepdims=True)
    edge_hist = jnp.sum(part[:, OFF_EDGE:OFF_EDGE + NUM_EDGE], axis=0, keepdims=True)
    val_hist = jnp.sum(part[:, OFF_VAL:OFF_VAL + VAL_LEN], axis=0, keepdims=True)

    k = jnp.arange(MAXN + 1).reshape(1, MAXN + 1).astype(jnp.float32)
    total_pairs = jnp.sum(n_hist * (k * (k - 1.0) * 0.5))
    bin0 = total_pairs - jnp.sum(edge_hist)
    edge_hist = edge_hist + bin0 * (
        jnp.arange(NUM_EDGE).reshape(1, NUM_EDGE) == 0
    ).astype(jnp.float32)

    n_dist = n_hist / jnp.sum(n_hist)
    node_dist = node_hist / jnp.sum(node_hist)
    edge_dist = edge_hist / jnp.sum(edge_hist)
    val_dist = val_hist / jnp.sum(val_hist)

    n_out[...] = n_dist
    node_out[...] = node_dist
    edge_out[...] = edge_dist
    val_out[...] = val_dist

    nt = nt_ref[...]
    nt = nt / jnp.sum(nt)
    node_t = node_t_ref[...]
    node_t = node_t / jnp.sum(node_t)
    et = et_ref[...]
    et = et / jnp.sum(et)

    nmae_out[...] = jnp.mean(jnp.abs(n_dist - nt)).reshape(1, 1)
    node_mae_out[...] = jnp.mean(jnp.abs(node_dist - node_t)).reshape(1, 1)
    edge_mae_out[...] = jnp.mean(jnp.abs(edge_dist - et)).reshape(1, 1)


def _finish(partials, n_target_dist, node_target_dist, edge_target_dist):
    f32 = jnp.float32
    return pl.pallas_call(
        _finish_body,
        out_shape=(
            jax.ShapeDtypeStruct((1, MAXN + 1), f32),
            jax.ShapeDtypeStruct((1, NUM_ATOM), f32),
            jax.ShapeDtypeStruct((1, NUM_EDGE), f32),
            jax.ShapeDtypeStruct((1, VAL_LEN), f32),
            jax.ShapeDtypeStruct((1, 1), f32),
            jax.ShapeDtypeStruct((1, 1), f32),
            jax.ShapeDtypeStruct((1, 1), f32),
        ),
    )(
        partials,
        n_target_dist.reshape(1, MAXN + 1),
        node_target_dist.reshape(1, NUM_ATOM),
        edge_target_dist.reshape(1, NUM_EDGE),
    )


def kernel(atom_types, edge_types, n_nodes,
           n_target_dist, node_target_dist, edge_target_dist):
    edge_t = jnp.transpose(edge_types, (1, 2, 0))
    atom_t = jnp.transpose(atom_types, (1, 0))
    partials = _sc_histograms(atom_t, edge_t, n_nodes)
    n_dist, node_dist, edge_dist, val_dist, n_mae, node_mae, edge_mae = _finish(
        partials, n_target_dist, node_target_dist, edge_target_dist
    )
    return (
        n_dist.reshape(MAXN + 1),
        node_dist.reshape(NUM_ATOM),
        edge_dist.reshape(NUM_EDGE),
        val_dist.reshape(VAL_LEN),
        n_mae.reshape(()),
        node_mae.reshape(()),
        edge_mae.reshape(()),
    )


# speedup vs baseline: 1.4282x; 1.4282x over previous
import functools

import jax
import jax.numpy as jnp
from jax import lax
from jax.experimental import pallas as pl
from jax.experimental.pallas import tpu as pltpu
from jax.experimental.pallas import tpu_sc as plsc

B = 8192
N = 64
NUM_ATOM = 16
NUM_EDGE = 5
MAXN = 64
VAL_LEN = 3 * MAXN - 2

OFF_N = 0
OFF_NODE = 80
OFF_EDGE = 96
OFF_VAL = 112
HTOT = 304

NC = 2
NS = 16
NW = NC * NS
MPW = B // NW
L = 16
LB = 128
JW = 8


def _sc_histograms(atom_t, edge_t, n_nodes):
    mesh = plsc.VectorSubcoreMesh(
        core_axis_name="c", subcore_axis_name="s", num_cores=NC, num_subcores=NS
    )

    @functools.partial(
        pl.kernel,
        mesh=mesh,
        compiler_params=pltpu.CompilerParams(needs_layout_passes=False),
        out_type=jax.ShapeDtypeStruct((NW, HTOT), jnp.float32),
        scratch_types=[
            pltpu.VMEM((MPW,), jnp.int32),
            pltpu.VMEM((N, LB), jnp.int32),
            pltpu.VMEM((N, JW, LB), jnp.int32),
            pltpu.VMEM((HTOT,), jnp.float32),
        ],
    )
    def body(atom_hbm, edge_hbm, n_hbm, out_hbm, nbuf, abuf, ebuf, hist):
        wid = lax.axis_index("s") * NC + lax.axis_index("c")
        base = wid * MPW

        zf = jnp.zeros((L,), jnp.float32)
        for h in range(HTOT // L):
            hist[pl.ds(h * L, L)] = zf

        pltpu.sync_copy(n_hbm.at[pl.ds(base, MPW)], nbuf)

        iota = lax.iota(jnp.int32, L)
        onesf = jnp.ones((L,), jnp.float32)
        zi = jnp.zeros((L,), jnp.int32)

        def ngrp(g, c):
            nv = nbuf[pl.ds(g * L, L)]
            plsc.addupdate_scatter(hist, [nv + OFF_N], onesf)
            return c
        lax.fori_loop(0, MPW // L, ngrp, 0)

        for mg in range(MPW // LB):
            mb = base + mg * LB

            pltpu.sync_copy(atom_hbm.at[:, pl.ds(mb, LB)], abuf)
            for sg in range(LB // L):
                n_vec = nbuf[pl.ds(mg * LB + sg * L, L)]
                maxn = jnp.max(n_vec)

                def node_row(i, cc, sg=sg, n_vec=n_vec):
                    av = abuf[i, pl.ds(sg * L, L)]
                    plsc.addupdate_scatter(
                        hist, [av + OFF_NODE], onesf, mask=i < n_vec
                    )
                    return cc
                lax.fori_loop(0, maxn, node_row, 0)

            def jslab(jc, c, mb=mb, mg=mg):
                pltpu.sync_copy(
                    edge_hbm.at[:, pl.ds(jc * JW, JW), pl.ds(mb, LB)], ebuf
                )
                fb = jnp.zeros((L,), jnp.bool_)
                for sg in range(LB // L):
                    n_vec = nbuf[pl.ds(mg * LB + sg * L, L)]
                    maxn = jnp.max(n_vec)

                    @pl.when(jc * JW < maxn)
                    def _(sg=sg, n_vec=n_vec, maxn=maxn, jc=jc):
                        j0 = jc * JW
                        mjs = [j0 + jl < n_vec for jl in range(JW)]

                        def irow(i, vas, band):
                            mi = i < n_vec
                            out = []
                            for jl in range(JW):
                                v = ebuf[i, jl, pl.ds(sg * L, L)]
                                etv = jnp.where(v == 4, 1, v)
                                pm = mi & mjs[jl]
                                if band == "tri":
                                    em = pm
                                elif band == "mix":
                                    em = jnp.where(j0 + jl > i, pm, fb)
                                else:
                                    em = None
                                if em is not None:
                                    plsc.addupdate_scatter(
                                        hist, [v + OFF_EDGE], onesf, mask=em
                                    )
                                out.append(vas[jl] + jnp.where(pm, etv, zi))
                            return tuple(out)

                        i_tri = jnp.minimum(j0, maxn)
                        i_mix = jnp.minimum(j0 + JW, maxn)
                        vas = lax.fori_loop(
                            0, i_tri,
                            lambda i, va: irow(i, va, "tri"), (zi,) * JW)
                        vas = lax.fori_loop(
                            i_tri, i_mix,
                            lambda i, va: irow(i, va, "mix"), vas)
                        vas = lax.fori_loop(
                            i_mix, maxn,
                            lambda i, va: irow(i, va, "val"), vas)
                        for jl in range(JW):
                            vc = jnp.minimum(vas[jl], VAL_LEN - 1) + OFF_VAL
                            plsc.addupdate_scatter(
                                hist, [vc], onesf, mask=mjs[jl]
                            )
                return c

            lax.fori_loop(0, N // JW, jslab, 0)

        pltpu.sync_copy(hist, out_hbm.at[wid])

    return body(atom_t, edge_t, n_nodes)


def _finish_body(part_ref, nt_ref, node_t_ref, et_ref,
                 n_out, node_out, edge_out, val_out,
                 nmae_out, node_mae_out, edge_mae_out):
    part = part_ref[...]

    n_hist = jnp.sum(part[:, OFF_N:OFF_N + MAXN + 1], axis=0, keepdims=True)
    node_hist = jnp.sum(part[:, OFF_NODE:OFF_NODE + NUM_ATOM], axis=0, keepdims=True)
    edge_hist = jnp.sum(part[:, OFF_EDGE:OFF_EDGE + NUM_EDGE], axis=0, keepdims=True)
    val_hist = jnp.sum(part[:, OFF_VAL:OFF_VAL + VAL_LEN], axis=0, keepdims=True)

    n_dist = n_hist / jnp.sum(n_hist)
    node_dist = node_hist / jnp.sum(node_hist)
    edge_dist = edge_hist / jnp.sum(edge_hist)
    val_dist = val_hist / jnp.sum(val_hist)

    n_out[...] = n_dist
    node_out[...] = node_dist
    edge_out[...] = edge_dist
    val_out[...] = val_dist

    nt = nt_ref[...]
    nt = nt / jnp.sum(nt)
    node_t = node_t_ref[...]
    node_t = node_t / jnp.sum(node_t)
    et = et_ref[...]
    et = et / jnp.sum(et)

    nmae_out[...] = jnp.mean(jnp.abs(n_dist - nt)).reshape(1, 1)
    node_mae_out[...] = jnp.mean(jnp.abs(node_dist - node_t)).reshape(1, 1)
    edge_mae_out[...] = jnp.mean(jnp.abs(edge_dist - et)).reshape(1, 1)


def _finish(partials, n_target_dist, node_target_dist, edge_target_dist):
    f32 = jnp.float32
    return pl.pallas_call(
        _finish_body,
        out_shape=(
            jax.ShapeDtypeStruct((1, MAXN + 1), f32),
            jax.ShapeDtypeStruct((1, NUM_ATOM), f32),
            jax.ShapeDtypeStruct((1, NUM_EDGE), f32),
            jax.ShapeDtypeStruct((1, VAL_LEN), f32),
            jax.ShapeDtypeStruct((1, 1), f32),
            jax.ShapeDtypeStruct((1, 1), f32),
            jax.ShapeDtypeStruct((1, 1), f32),
        ),
    )(
        partials,
        n_target_dist.reshape(1, MAXN + 1),
        node_target_dist.reshape(1, NUM_ATOM),
        edge_target_dist.reshape(1, NUM_EDGE),
    )


def kernel(atom_types, edge_types, n_nodes,
           n_target_dist, node_target_dist, edge_target_dist):
    edge_t = jnp.transpose(edge_types, (1, 2, 0))
    atom_t = jnp.transpose(atom_types, (1, 0))
    partials = _sc_histograms(atom_t, edge_t, n_nodes)
    n_dist, node_dist, edge_dist, val_dist, n_mae, node_mae, edge_mae = _finish(
        partials, n_target_dist, node_target_dist, edge_target_dist
    )
    return (
        n_dist.reshape(MAXN + 1),
        node_dist.reshape(NUM_ATOM),
        edge_dist.reshape(NUM_EDGE),
        val_dist.reshape(VAL_LEN),
        n_mae.reshape(()),
        node_mae.reshape(()),
        edge_mae.reshape(()),
    )


# speedup vs baseline: 1.4999x; 1.0502x over previous
import functools

import jax
import jax.numpy as jnp
from jax import lax
from jax.experimental import pallas as pl
from jax.experimental.pallas import tpu as pltpu
from jax.experimental.pallas import tpu_sc as plsc

B = 8192
N = 64
NUM_ATOM = 16
NUM_EDGE = 5
MAXN = 64
VAL_LEN = 3 * MAXN - 2

OFF_N = 0
OFF_NODE = 80
OFF_EDGE = 96
OFF_VAL = 176
HTOT = 368

NC = 2
NS = 16
NW = NC * NS
MPW = B // NW
L = 16
LB = 128
JW = 8


def _sc_histograms(atom_t, edge_t, n_nodes):
    mesh = plsc.VectorSubcoreMesh(
        core_axis_name="c", subcore_axis_name="s", num_cores=NC, num_subcores=NS
    )

    @functools.partial(
        pl.kernel,
        mesh=mesh,
        compiler_params=pltpu.CompilerParams(needs_layout_passes=False),
        out_type=jax.ShapeDtypeStruct((NW, HTOT), jnp.float32),
        scratch_types=[
            pltpu.VMEM((MPW,), jnp.int32),
            pltpu.VMEM((N, LB), jnp.int32),
            pltpu.VMEM((N, JW, LB), jnp.int32),
            pltpu.VMEM((HTOT,), jnp.float32),
        ],
    )
    def body(atom_hbm, edge_hbm, n_hbm, out_hbm, nbuf, abuf, ebuf, hist):
        wid = lax.axis_index("s") * NC + lax.axis_index("c")
        base = wid * MPW

        zf = jnp.zeros((L,), jnp.float32)
        for h in range(HTOT // L):
            hist[pl.ds(h * L, L)] = zf

        pltpu.sync_copy(n_hbm.at[pl.ds(base, MPW)], nbuf)

        iota = lax.iota(jnp.int32, L)
        onesf = jnp.ones((L,), jnp.float32)
        zi = jnp.zeros((L,), jnp.int32)

        def ngrp(g, c):
            nv = nbuf[pl.ds(g * L, L)]
            plsc.addupdate_scatter(hist, [nv + OFF_N], onesf)
            return c
        lax.fori_loop(0, MPW // L, ngrp, 0)

        for mg in range(MPW // LB):
            mb = base + mg * LB

            pltpu.sync_copy(atom_hbm.at[:, pl.ds(mb, LB)], abuf)
            for sg in range(LB // L):
                n_vec = nbuf[pl.ds(mg * LB + sg * L, L)]
                maxn = jnp.max(n_vec)

                def node_row(i, cc, sg=sg, n_vec=n_vec):
                    av = abuf[i, pl.ds(sg * L, L)]
                    plsc.addupdate_scatter(
                        hist, [av + OFF_NODE], onesf, mask=i < n_vec
                    )
                    return cc
                lax.fori_loop(0, maxn, node_row, 0)

            def jslab(jc, c, mb=mb, mg=mg):
                pltpu.sync_copy(
                    edge_hbm.at[:, pl.ds(jc * JW, JW), pl.ds(mb, LB)], ebuf
                )
                fb = jnp.zeros((L,), jnp.bool_)
                for sg in range(LB // L):
                    n_vec = nbuf[pl.ds(mg * LB + sg * L, L)]
                    maxn = jnp.max(n_vec)

                    @pl.when(jc * JW < maxn)
                    def _(sg=sg, n_vec=n_vec, maxn=maxn, jc=jc):
                        j0 = jc * JW
                        mjs = [j0 + jl < n_vec for jl in range(JW)]

                        def irow(i, vas, band):
                            mi = i < n_vec
                            out = []
                            for jl in range(JW):
                                v = ebuf[i, jl, pl.ds(sg * L, L)]
                                etv = jnp.where(v == 4, 1, v)
                                pm = mi & mjs[jl]
                                if band == "tri":
                                    em = pm
                                elif band == "mix":
                                    em = jnp.where(j0 + jl > i, pm, fb)
                                else:
                                    em = None
                                if em is not None:
                                    plsc.addupdate_scatter(
                                        hist,
                                        [(v << 4) + (iota + OFF_EDGE)],
                                        onesf, mask=em,
                                    )
                                out.append(vas[jl] + jnp.where(pm, etv, zi))
                            return tuple(out)

                        i_tri = jnp.minimum(j0, maxn)
                        i_mix = jnp.minimum(j0 + JW, maxn)
                        vas = lax.fori_loop(
                            0, i_tri,
                            lambda i, va: irow(i, va, "tri"), (zi,) * JW)
                        vas = lax.fori_loop(
                            i_tri, i_mix,
                            lambda i, va: irow(i, va, "mix"), vas)
                        vas = lax.fori_loop(
                            i_mix, maxn,
                            lambda i, va: irow(i, va, "val"), vas)
                        for jl in range(JW):
                            vc = jnp.minimum(vas[jl], VAL_LEN - 1) + OFF_VAL
                            plsc.addupdate_scatter(
                                hist, [vc], onesf, mask=mjs[jl]
                            )
                return c

            lax.fori_loop(0, N // JW, jslab, 0)

        pltpu.sync_copy(hist, out_hbm.at[wid])

    return body(atom_t, edge_t, n_nodes)


def _finish_body(part_ref, nt_ref, node_t_ref, et_ref,
                 n_out, node_out, edge_out, val_out,
                 nmae_out, node_mae_out, edge_mae_out):
    part = part_ref[...]

    n_hist = jnp.sum(part[:, OFF_N:OFF_N + MAXN + 1], axis=0, keepdims=True)
    node_hist = jnp.sum(part[:, OFF_NODE:OFF_NODE + NUM_ATOM], axis=0, keepdims=True)
    eidx = jnp.arange(NUM_EDGE).reshape(1, NUM_EDGE)
    edge_hist = jnp.zeros((1, NUM_EDGE), jnp.float32)
    for e in range(NUM_EDGE):
        s_e = jnp.sum(part[:, OFF_EDGE + 16 * e:OFF_EDGE + 16 * (e + 1)])
        edge_hist = edge_hist + s_e * (eidx == e).astype(jnp.float32)
    val_hist = jnp.sum(part[:, OFF_VAL:OFF_VAL + VAL_LEN], axis=0, keepdims=True)

    n_dist = n_hist / jnp.sum(n_hist)
    node_dist = node_hist / jnp.sum(node_hist)
    edge_dist = edge_hist / jnp.sum(edge_hist)
    val_dist = val_hist / jnp.sum(val_hist)

    n_out[...] = n_dist
    node_out[...] = node_dist
    edge_out[...] = edge_dist
    val_out[...] = val_dist

    nt = nt_ref[...]
    nt = nt / jnp.sum(nt)
    node_t = node_t_ref[...]
    node_t = node_t / jnp.sum(node_t)
    et = et_ref[...]
    et = et / jnp.sum(et)

    nmae_out[...] = jnp.mean(jnp.abs(n_dist - nt)).reshape(1, 1)
    node_mae_out[...] = jnp.mean(jnp.abs(node_dist - node_t)).reshape(1, 1)
    edge_mae_out[...] = jnp.mean(jnp.abs(edge_dist - et)).reshape(1, 1)


def _finish(partials, n_target_dist, node_target_dist, edge_target_dist):
    f32 = jnp.float32
    return pl.pallas_call(
        _finish_body,
        out_shape=(
            jax.ShapeDtypeStruct((1, MAXN + 1), f32),
            jax.ShapeDtypeStruct((1, NUM_ATOM), f32),
            jax.ShapeDtypeStruct((1, NUM_EDGE), f32),
            jax.ShapeDtypeStruct((1, VAL_LEN), f32),
            jax.ShapeDtypeStruct((1, 1), f32),
            jax.ShapeDtypeStruct((1, 1), f32),
            jax.ShapeDtypeStruct((1, 1), f32),
        ),
    )(
        partials,
        n_target_dist.reshape(1, MAXN + 1),
        node_target_dist.reshape(1, NUM_ATOM),
        edge_target_dist.reshape(1, NUM_EDGE),
    )


def kernel(atom_types, edge_types, n_nodes,
           n_target_dist, node_target_dist, edge_target_dist):
    edge_t = jnp.transpose(edge_types, (1, 2, 0))
    atom_t = jnp.transpose(atom_types, (1, 0))
    partials = _sc_histograms(atom_t, edge_t, n_nodes)
    n_dist, node_dist, edge_dist, val_dist, n_mae, node_mae, edge_mae = _finish(
        partials, n_target_dist, node_target_dist, edge_target_dist
    )
    return (
        n_dist.reshape(MAXN + 1),
        node_dist.reshape(NUM_ATOM),
        edge_dist.reshape(NUM_EDGE),
        val_dist.reshape(VAL_LEN),
        n_mae.reshape(()),
        node_mae.reshape(()),
        edge_mae.reshape(()),
    )
